# baseline (speedup 1.0000x reference)
import jax
import jax.numpy as jnp
from jax.experimental import pallas as pl
from jax.experimental.pallas import tpu as pltpu
from jax.experimental.pallas import tpu_sc as plsc

NBC = 8192
CD = 64
WID = 512
DEP = 3
DNT = 3
IND = 263
BB = 32
TT = 64
F32 = jnp.float32

_PREC = jax.lax.Precision.DEFAULT


def _dot(a, b, prec=_PREC):
    return jax.lax.dot_general(
        a, b, (((1,), (0,)), ((), ())),
        precision=prec, preferred_element_type=F32)


def _shift(x, s, t_len):
    if s == 0:
        return x
    r, c = x.shape
    z = jnp.zeros((abs(s), c), x.dtype)
    if s > 0:
        y = jnp.concatenate([x[s:, :], z], axis=0)
    else:
        y = jnp.concatenate([z, x[:r + s, :]], axis=0)
    tpos = jax.lax.broadcasted_iota(jnp.int32, (r, 1), 0) & (t_len - 1)
    ok = (tpos + s >= 0) & (tpos + s < t_len)
    return jnp.where(ok, y, jnp.zeros_like(y))


def _conv3(x, wc, b, d, t_len):
    xc = jnp.concatenate(
        [_shift(x, -d, t_len), x, _shift(x, d, t_len)], axis=1)
    return _dot(xc, wc) + b


def _resblk(x, wc1, b1, w2, b2, d, t_len):
    h = jnp.maximum(x, 0.0)
    h = _conv3(h, wc1, b1, d, t_len)
    h = jnp.maximum(h, 0.0)
    return x + _dot(h, w2) + b2


def _enc_in_body(x_ref, wc, b, o_ref):
    o_ref[...] = jnp.maximum(
        _conv3(x_ref[...], wc[...], b[...], 1, TT), 0.0)


def _mk_down_block(t_out, with_enc_out):
    def body(*refs):
        o_ref = refs[-1]
        it = iter(refs[:-1])
        nxt = lambda: next(it)[...]
        he, ho = nxt(), nxt()
        wc4, b4 = nxt(), nxt()
        xc = jnp.concatenate(
            [_shift(ho, -1, t_out), he, ho, _shift(he, 1, t_out)], axis=1)
        h = _dot(xc, wc4) + b4
        for d in (1, 3, 9):
            h = _resblk(h, nxt(), nxt(), nxt(), nxt(), d, t_out)
        if with_enc_out:
            h = _conv3(h, nxt(), nxt(), 1, t_out)
        o_ref[...] = h
    return body


def _mk_up_block(t_in, with_dec_in):
    def body(*refs):
        oe_ref, oo_ref = refs[-2], refs[-1]
        it = iter(refs[:-2])
        nxt = lambda: next(it)[...]
        g = nxt()
        if with_dec_in:
            g = jnp.maximum(_conv3(g, nxt(), nxt(), 1, t_in), 0.0)
        for d in (9, 3, 1):
            g = _resblk(g, nxt(), nxt(), nxt(), nxt(), d, t_in)
        wc3, b3 = nxt(), nxt()
        xe = jnp.concatenate([_shift(g, -1, t_in), g, g], axis=1)
        xo = jnp.concatenate([g, g, _shift(g, 1, t_in)], axis=1)
        oe_ref[...] = _dot(xe, wc3) + b3
        oo_ref[...] = _dot(xo, wc3) + b3
    return body


def _dec_tail_body(g_ref, mwc, mb, owc, ob, o_ref):
    g = jnp.maximum(_conv3(g_ref[...], mwc[...], mb[...], 1, TT), 0.0)
    o_ref[...] = _conv3(g, owc[...], ob[...], 1, TT)


def _quant_body(xf_ref, cbt_ref, idx_ref, com_ref, per_ref):
    xf = xf_ref[...]
    cbt = cbt_ref[...]
    rows = xf.shape[0]
    dotm = _dot(xf, cbt, jax.lax.Precision.DEFAULT)
    xsq = jnp.sum(xf * xf, axis=1, keepdims=True)
    csq = jnp.sum(cbt * cbt, axis=0, keepdims=True)
    dist = xsq - 2.0 * dotm + csq
    mind = jnp.min(dist, axis=1, keepdims=True)
    col = jax.lax.broadcasted_iota(jnp.int32, dist.shape, 1)
    idx = jnp.min(jnp.where(dist == mind, col, NBC), axis=1, keepdims=True)
    idx_ref[...] = idx
    onehot = (col == idx).astype(F32)
    cnt = jnp.sum(onehot, axis=0, keepdims=True)
    probs = cnt * (1.0 / rows)
    ent = -jnp.sum(probs * jnp.log(probs + 1e-10))
    per_ref[...] = jnp.exp(ent).reshape(1, 1)
    com_ref[...] = (jnp.sum(mind) * (1.0 / (rows * CD))).reshape(1, 1)


def _pc(body, out_shapes, args):
    outs = [jax.ShapeDtypeStruct(s, d) for (s, d) in out_shapes]
    res = pl.pallas_call(body, out_shape=outs)(*args)
    return res[0] if len(out_shapes) == 1 else res


def _cmat(p):
    w = p["w"]
    o, i, k = w.shape
    return w.transpose(2, 1, 0).reshape(k * i, o), p["b"][None, :]


def _sc_gather(codebook, idx_flat, rows):
    mesh = plsc.VectorSubcoreMesh(core_axis_name="c", subcore_axis_name="s")
    win = 128
    vd = 128

    @pl.kernel(out_type=jax.ShapeDtypeStruct((rows, vd), F32), mesh=mesh)
    def gather_kernel(cb_hbm, i_hbm, o_hbm):
        def body(i_vmem, o_vmem):
            pltpu.sync_copy(cb_hbm.at[i_vmem.at[0]], o_vmem)

        pltpu.emit_pipeline(
            body,
            grid=(rows // win,),
            in_specs=[pl.BlockSpec((1, win), index_map=lambda i: (0, i))],
            out_specs=[pl.BlockSpec((win, vd), index_map=lambda i: (i, 0))],
            core_axis_name="s",
            dimension_semantics=(pltpu.PARALLEL,),
        )(i_hbm, o_hbm)

    cb_pad = jnp.pad(codebook, ((0, 0), (0, vd - CD)))
    return gather_kernel(cb_pad, idx_flat)[:, :CD]


def _encode(x, p):
    h = x.reshape(BB * TT, IND)
    ewc, eb = _cmat(p["enc_in"])
    h = _pc(_enc_in_body, [((BB * TT, WID), F32)], [h, ewc, eb])
    t = TT
    for bi, blk in enumerate(p["enc_down"]):
        he, ho = _phases(h, t, WID)
        t //= 2
        dwc, db = _cmat(blk["down"])
        args = [he, ho, dwc, db]
        for rp in blk["res"]:
            args += _res_args(rp)
        last = bi == DNT - 1
        if last:
            owc, ob = _cmat(p["enc_out"])
            args += [owc, ob]
        h = _pc(_mk_down_block(t, last),
                [((BB * t, CD if last else WID), F32)], args)
    return h


def _quantize(xf, codebook):
    rows = xf.shape[0]
    idx, com, per = _pc(
        _quant_body,
        [((rows, 1), jnp.int32), ((1, 1), F32), ((1, 1), F32)],
        [xf, codebook.T])
    xd = _sc_gather(codebook, idx.reshape(1, rows), rows)
    return idx, xd, com, per


def _decode(xd, p):
    g = xd
    t = TT // (2 ** DNT)
    for bi, blk in enumerate(p["dec_up"]):
        args = [g]
        if bi == 0:
            iwc, ib = _cmat(p["dec_in"])
            args += [iwc, ib]
        for rp in blk["res"]:
            args += _res_args(rp)
        uwc, ub = _cmat(blk["up"])
        args += [uwc, ub]
        ye, yo = _pc(_mk_up_block(t, bi == 0),
                     [((BB * t, WID), F32), ((BB * t, WID), F32)], args)
        g = _interleave(ye, yo, t)
        t *= 2
    mwc, mb = _cmat(p["dec_mid"])
    owc, ob = _cmat(p["dec_out"])
    out = _pc(_dec_tail_body, [((BB * TT, IND), F32)],
              [g, mwc, mb, owc, ob])
    return out.reshape(BB, TT, IND)


def _phases(h, t_in, c):
    h3 = h.reshape(BB, t_in, c)
    he = h3[:, 0::2, :].reshape(BB * (t_in // 2), c)
    ho = h3[:, 1::2, :].reshape(BB * (t_in // 2), c)
    return he, ho


def _interleave(ye, yo, t_half):
    y = jnp.stack([ye.reshape(BB, t_half, WID), yo.reshape(BB, t_half, WID)],
                  axis=2)
    return y.reshape(BB * t_half * 2, WID)


def _res_args(rp):
    wc1, b1 = _cmat(rp["c1"])
    w2 = rp["c2"]["w"][:, :, 0].T
    b2 = rp["c2"]["b"][None, :]
    return [wc1, b1, w2, b2]


def kernel(x, params, codebook):
    xf = _encode(x, params)
    idx, xd, com, per = _quantize(xf, codebook)
    x_out = _decode(xd, params)
    return (x_out, com[0, 0], jnp.float32(0.0), per[0, 0])
